# baseline (speedup 1.0000x reference)
import jax
import jax.numpy as jnp
from jax import lax
from jax.experimental import pallas as pl

_BR = 512
_BC = 4096
_N_ROWS = 16384
_N_COLS = 4096
_K = 128


def _tc_body(mask_ref, x_ref, out_ref):
    m = mask_ref[0, :]
    iota = lax.broadcasted_iota(jnp.int32, (_K, _BC), 0)
    onehot = (iota == m[None, :]).astype(jnp.float32)
    out_ref[...] = jnp.dot(x_ref[...], onehot,
                           preferred_element_type=jnp.float32)


def kernel(x, mask):
    return pl.pallas_call(
        _tc_body,
        grid=(_N_ROWS // _BR,),
        in_specs=[
            pl.BlockSpec((1, _BC), lambda i: (0, 0)),
            pl.BlockSpec((_BR, _K), lambda i: (i, 0)),
        ],
        out_specs=pl.BlockSpec((_BR, _BC), lambda i: (i, 0)),
        out_shape=jax.ShapeDtypeStruct((_N_ROWS, _N_COLS), jnp.float32),
    )(mask.reshape(1, _N_COLS), x)
